# Initial kernel scaffold; baseline (speedup 1.0000x reference)
import jax
import jax.numpy as jnp
from jax.experimental import pallas as pl


def kernel(x, edge_index, edge_type, pool_indices, node_types, W_rel, W_root, b):
    raise NotImplementedError("write your pallas kernel here")


# speedup vs baseline: 12.7493x; 12.7493x over previous
import functools

import jax
import jax.numpy as jnp
from jax import lax
from jax.experimental import pallas as pl
from jax.experimental.pallas import tpu as pltpu
from jax.experimental.pallas import tpu_sc as plsc

N = 10000
E = 320000
D = 128
R = 8
P = 2048

DP = 144
HALF = 1024
NC = 2
NS = 16
E_PER_TILE = E // NS
SB = 2000
ST = E_PER_TILE // SB
G = 80
GROUPS = SB // G
TRASH = R * HALF
A_SP_ROWS = 8320
ZROWS_PER_TILE = A_SP_ROWS // NS
AOUT_ROWS_PER_TILE = (R * HALF) // NS


def _pad_body(x_ref, o_ref):
    xb = x_ref[...]
    tail = (lax.broadcasted_iota(jnp.int32, (xb.shape[0], DP - D), 1) == 0)
    o_ref[...] = jnp.concatenate([xb, tail.astype(jnp.float32)], axis=1)


def _make_x_aug(x):
    bn = 1000
    return pl.pallas_call(
        _pad_body,
        grid=(N // bn,),
        in_specs=[pl.BlockSpec((bn, D), lambda i: (i, 0))],
        out_specs=pl.BlockSpec((bn, DP), lambda i: (i, 0)),
        out_shape=jax.ShapeDtypeStruct((N, DP), jnp.float32),
    )(x)


def _sc_body(x_hbm, xaug_hbm, src_hbm, dst_hbm, typ_hbm, pool_hbm, ntyp_hbm,
             zeros_hbm,
             a_out, xpool_out, slotent_out, went_out,
             slot_tab, pool_v, src_s, dst_s, typ_s,
             rows0, rows1, xrows, entbuf_i, entbuf_f, a_sp,
             sem0, sem1, semx):
    cid = lax.axis_index("c")
    sid = lax.axis_index("s")
    wid = sid * NC + cid
    base_slot = cid * HALF

    pltpu.sync_copy(pool_hbm, pool_v)

    def initbody(i, c):
        slot_tab[pl.ds(i * 16, 16)] = jnp.full((16,), -1, jnp.int32)
        return c
    lax.fori_loop(0, N // 16, initbody, 0)

    iota16 = lax.broadcasted_iota(jnp.int32, (16,), 0)

    def scatbody(i, c):
        pv = pool_v[pl.ds(i * 16, 16)]
        plsc.store_scatter(slot_tab, [pv], iota16 + i * 16)
        return c
    lax.fori_loop(0, P // 16, scatbody, 0)

    pltpu.sync_copy(zeros_hbm.at[pl.ds(sid * ZROWS_PER_TILE, ZROWS_PER_TILE)],
                    a_sp.at[pl.ds(sid * ZROWS_PER_TILE, ZROWS_PER_TILE)])
    plsc.subcore_barrier()

    for st in range(ST):
        ebase = sid * E_PER_TILE + st * SB
        pltpu.sync_copy(src_hbm.at[pl.ds(ebase, SB)], src_s)
        pltpu.sync_copy(dst_hbm.at[pl.ds(ebase, SB)], dst_s)
        pltpu.sync_copy(typ_hbm.at[pl.ds(ebase, SB)], typ_s)

        pltpu.async_copy(xaug_hbm.at[src_s.at[pl.ds(0, G)]], rows0, sem0)
        pltpu.async_copy(xaug_hbm.at[src_s.at[pl.ds(G, G)]], rows1, sem1)

        def pairbody(j, c):
            for par, rb, sem in ((0, rows0, sem0), (1, rows1, sem1)):
                g = 2 * j + par
                pltpu.make_async_copy(xaug_hbm.at[pl.ds(0, G)], rb, sem).wait()
                for k in range(G // 16):
                    off = g * G + k * 16
                    dstv = dst_s[pl.ds(off, 16)]
                    typv = typ_s[pl.ds(off, 16)]
                    sl = plsc.load_gather(slot_tab, [dstv])
                    loc = sl - base_slot
                    valid = (loc >= 0) & (loc < HALF)
                    arow = jnp.where(valid, typv * HALF + loc,
                                     jnp.full((16,), TRASH, jnp.int32))
                    pltpu.sync_copy(rb.at[pl.ds(k * 16, 16)],
                                    a_sp.at[arow], add=True)

                @pl.when(j < GROUPS // 2 - 1)
                def _():
                    pltpu.async_copy(
                        xaug_hbm.at[src_s.at[pl.ds((g + 2) * G, G)]], rb, sem)
            return c
        lax.fori_loop(0, GROUPS // 2, pairbody, 0)

    plsc.subcore_barrier()

    typ_idx = sid // 2
    loc_start = (sid % 2) * AOUT_ROWS_PER_TILE
    pltpu.sync_copy(
        a_sp.at[pl.ds(sid * AOUT_ROWS_PER_TILE, AOUT_ROWS_PER_TILE)],
        a_out.at[typ_idx, pl.ds(cid * HALF + loc_start, AOUT_ROWS_PER_TILE)])

    xb = wid * (P // (NC * NS))
    pltpu.async_copy(x_hbm.at[pool_v.at[pl.ds(xb, P // (NC * NS))]], xrows,
                     semx).wait()
    pltpu.sync_copy(xrows, xpool_out.at[pl.ds(xb, P // (NC * NS))])

    @pl.when((cid == 0) & (sid == 0))
    def _():
        def sebody(i, c):
            pv = pool_v[pl.ds(i * 16, 16)]
            se = plsc.load_gather(slot_tab, [pv])
            entbuf_i[pl.ds(i * 16, 16)] = se
            return c
        lax.fori_loop(0, P // 16, sebody, 0)
        pltpu.sync_copy(entbuf_i, slotent_out)

        pltpu.sync_copy(ntyp_hbm, slot_tab)

        def wbody(i, c):
            pv = pool_v[pl.ds(i * 16, 16)]
            nt = plsc.load_gather(slot_tab, [pv])
            w = jnp.where(nt == 0, jnp.full((16,), 4.0, jnp.float32),
                          jnp.where(nt == 1, jnp.full((16,), 1.0, jnp.float32),
                                    jnp.full((16,), 2.0, jnp.float32)))
            entbuf_f[pl.ds(i * 16, 16)] = w
            return c
        lax.fori_loop(0, P // 16, wbody, 0)
        pltpu.sync_copy(entbuf_f, went_out)


def _sc_accumulate(x, x_aug, src, dst, typ, pool, ntypes, zeros):
    mesh = plsc.VectorSubcoreMesh(core_axis_name="c", subcore_axis_name="s")
    fn = pl.kernel(
        _sc_body,
        out_type=(
            jax.ShapeDtypeStruct((R, P, DP), jnp.float32),
            jax.ShapeDtypeStruct((P, D), jnp.float32),
            jax.ShapeDtypeStruct((P,), jnp.int32),
            jax.ShapeDtypeStruct((P,), jnp.float32),
        ),
        mesh=mesh,
        compiler_params=pltpu.CompilerParams(use_tc_tiling_on_sc=False,
                                             needs_layout_passes=False),
        scratch_types=[
            pltpu.VMEM((N,), jnp.int32),
            pltpu.VMEM((P,), jnp.int32),
            pltpu.VMEM((SB,), jnp.int32),
            pltpu.VMEM((SB,), jnp.int32),
            pltpu.VMEM((SB,), jnp.int32),
            pltpu.VMEM((G, DP), jnp.float32),
            pltpu.VMEM((G, DP), jnp.float32),
            pltpu.VMEM((P // (NC * NS), D), jnp.float32),
            pltpu.VMEM((P,), jnp.int32),
            pltpu.VMEM((P,), jnp.float32),
            pltpu.VMEM_SHARED((A_SP_ROWS, DP), jnp.float32),
            pltpu.SemaphoreType.DMA,
            pltpu.SemaphoreType.DMA,
            pltpu.SemaphoreType.DMA,
        ],
    )
    return fn(x, x_aug, src, dst, typ, pool, ntypes, zeros)


def _tc_body(a_ref, xp_ref, se_ref, we_ref, wr_ref, wroot_ref, b_ref, out_ref):
    hi = jax.lax.Precision.HIGHEST
    xp = xp_ref[...]
    acc = jnp.dot(xp, wroot_ref[...], precision=hi)
    agg = jnp.zeros((P, D), jnp.float32)
    deg = jnp.zeros((P, 1), jnp.float32)
    for r in range(R):
        ar = a_ref[r]
        agg = agg + jnp.dot(ar[:, :D], wr_ref[r], precision=hi)
        deg = deg + jnp.sum(ar[:, D:DP], axis=1, keepdims=True)
    emb = jnp.maximum(agg / jnp.maximum(deg, 1.0) + acc + b_ref[...], 0.0)

    se = se_ref[...]
    we = we_ref[...]
    ws_parts = []
    bs = 256
    for blk in range(P // bs):
        iota_blk = lax.broadcasted_iota(jnp.int32, (P, bs), 1) + blk * bs
        m = jnp.where(se == iota_blk, we, 0.0)
        ws_parts.append(jnp.sum(m, axis=0, keepdims=True))
    ws = jnp.concatenate(ws_parts, axis=1)
    num = jnp.dot(ws, emb, precision=hi)
    den = jnp.sum(we) + 1e-9
    out_ref[...] = num / den


def _tc_finish(a, xpool, slotent, went, w_rel, w_root, b):
    return pl.pallas_call(
        _tc_body,
        out_shape=jax.ShapeDtypeStruct((1, D), jnp.float32),
    )(a, xpool, slotent.reshape(P, 1), went.reshape(P, 1), w_rel, w_root,
      b.reshape(1, D))


def kernel(x, edge_index, edge_type, pool_indices, node_types, W_rel, W_root,
           b):
    src = edge_index[0]
    dst = edge_index[1]
    x_aug = _make_x_aug(x)
    zeros = jnp.zeros((A_SP_ROWS, DP), jnp.float32)
    a, xpool, slotent, went = _sc_accumulate(
        x, x_aug, src, dst, edge_type, pool_indices, node_types, zeros)
    return _tc_finish(a, xpool, slotent, went, W_rel, W_root, b)
